# baseline (speedup 1.0000x reference)
import functools
import math

import numpy as np
import jax
import jax.numpy as jnp
from jax import lax
from jax.experimental import pallas as pl
from jax.experimental.pallas import tpu as pltpu
from jax.experimental.pallas import tpu_sc as plsc

NUM_CASTES = 5
NUM_NODES = 11
EMB = 128
HID = 64
H = 2
D = HID // H

_NODE_INDEX = {'atp_executor': 0, 'order_tracking': 1, 'po_creation': 2, 'rebalancing': 3, 'subcontracting': 4, 'safety_stock': 5, 'forecast_adj': 6, 'quality': 7, 'maintenance': 8, 'mo_execution': 9, 'to_execution': 10}
_TO_CASTE = {'atp_executor': 0, 'order_tracking': 0, 'po_creation': 1, 'rebalancing': 1, 'subcontracting': 1, 'safety_stock': 2, 'forecast_adj': 2, 'quality': 3, 'maintenance': 3, 'mo_execution': 4, 'to_execution': 4}
_EDGES = [('atp_executor','po_creation'),('atp_executor','rebalancing'),('atp_executor','subcontracting'),('atp_executor','safety_stock'),('order_tracking','po_creation'),('po_creation','atp_executor'),('po_creation','order_tracking'),('rebalancing','atp_executor'),('rebalancing','to_execution'),('subcontracting','mo_execution'),('safety_stock','po_creation'),('safety_stock','atp_executor'),('forecast_adj','po_creation'),('forecast_adj','safety_stock'),('forecast_adj','atp_executor'),('quality','atp_executor'),('quality','mo_execution'),('maintenance','mo_execution'),('maintenance','subcontracting'),('mo_execution','atp_executor'),('mo_execution','po_creation'),('mo_execution','subcontracting'),('to_execution','order_tracking'),('to_execution','rebalancing')]

_SRC = np.array([_NODE_INDEX[s] for s, t in _EDGES], dtype=np.int32)
_TGT = np.array([_NODE_INDEX[t] for s, t in _EDGES], dtype=np.int32)
_ETYPE = np.array([_TO_CASTE[s] * NUM_CASTES + _TO_CASTE[t] for s, t in _EDGES], dtype=np.int32)
_CASTES = np.array([_TO_CASTE[n] for n in sorted(_NODE_INDEX, key=lambda k: _NODE_INDEX[k])], dtype=np.int32)
NE = len(_EDGES)

_PERM = np.argsort(_TGT, kind='stable')
_SRCP = _SRC[_PERM]
_TGTP = _TGT[_PERM]
_ETP = _ETYPE[_PERM]
_SEGS = []
_i = 0
while _i < NE:
    _j = _i
    while _j < NE and _TGTP[_j] == _TGTP[_i]:
        _j += 1
    _SEGS.append((int(_i), int(_j)))
    _i = _j
_HAS_IN = np.zeros(NUM_NODES, dtype=np.float32)
_HAS_IN[np.unique(_TGTP)] = 1.0
_T2SEG = {int(_TGTP[a]): a for (a, b) in reversed(_SEGS)}

NSC = 2 * NE
NW = 2 * NSC
NOUT = NUM_NODES * HID


def _prep(W_node, b_node, caste_table, Wq, bq, Wk, bk, Wv, bv,
          edge_bias_table, Wo, bo, gamma, beta):
    f32 = jnp.float32
    W_state = W_node[:EMB].astype(f32)
    w_u = W_node[EMB].astype(f32)
    W_caste = W_node[EMB + 1:].astype(f32)
    cnode = caste_table[_CASTES] @ W_caste + b_node
    Qc = (cnode @ Wq + bq).reshape(NUM_NODES, H, D)
    Kc = (cnode @ Wk + bk).reshape(NUM_NODES, H, D)
    Vc = (cnode @ Wv + bv).reshape(NUM_NODES, H, D)
    qu = (w_u @ Wq).reshape(H, D)
    ku = (w_u @ Wk).reshape(H, D)
    vu = (w_u @ Wv).reshape(H, D)
    WQ = (W_state @ Wq).reshape(EMB, H, D)
    inv = 1.0 / math.sqrt(D)

    aW = jnp.einsum('khd,hd->kh', WQ, ku) * inv
    gW = (jnp.einsum('khd,shd->khs', WQ, Kc) * inv).reshape(EMB, NUM_NODES * H)
    C1 = jnp.einsum('hd,hd->h', qu, ku) * inv
    C2 = jnp.einsum('hd,shd->sh', qu, Kc) * inv
    C3 = jnp.einsum('thd,hd->th', Qc, ku) * inv
    C5 = jnp.einsum('thd,shd->tsh', Qc, Kc) * inv
    C4 = C5[_TGTP, _SRCP, :] + edge_bias_table[_ETP]

    Woh = Wo.reshape(H, D, HID)
    P = jnp.einsum('hd,hdo->ho', vu, Woh)
    Rm = jnp.einsum('shd,hdo->sho', Vc, Woh)
    VbWo = W_state @ Wv @ Wo

    WT704 = jnp.concatenate(
        [W_state + _HAS_IN[t] * VbWo for t in range(NUM_NODES)], axis=1)
    WGA = jnp.concatenate([gW, aW], axis=1)

    tgt_oh = np.zeros((NE, NUM_NODES), dtype=np.float32)
    tgt_oh[np.arange(NE), _TGTP] = 1.0
    tgt_oh = jnp.asarray(tgt_oh)
    S_w = jnp.concatenate([
        jnp.einsum('et,eo->eto', tgt_oh, Rm[_SRCP, h]).reshape(NE, NOUT)
        for h in range(H)], axis=0)
    S_w2 = jnp.concatenate([
        jnp.einsum('et,o->eto', tgt_oh, P[h]).reshape(NE, NOUT)
        for h in range(H)], axis=0)

    UW = jnp.einsum('tu,o->tuo', jnp.eye(NUM_NODES, dtype=f32), w_u).reshape(NUM_NODES, NOUT)

    EUst = np.zeros((NUM_NODES, 2 * NSC), dtype=np.float32)
    MG = np.zeros((NUM_NODES * H + H, NSC), dtype=np.float32)
    MA = np.zeros((NUM_NODES * H + H, NSC), dtype=np.float32)
    Gmat = np.zeros((NSC, NSC), dtype=np.float32)
    for h in range(H):
        for e in range(NE):
            c = h * NE + e
            EUst[_SRCP[e], c] = 1.0
            EUst[_TGTP[e], NSC + c] = 1.0
            MG[h * NUM_NODES + _SRCP[e], c] = 1.0
            MA[NUM_NODES * H + h, c] = 1.0
        for (a, b) in _SEGS:
            for e in range(a, b):
                for e2 in range(a, b):
                    Gmat[h * NE + e, h * NE + e2] = 1.0
    EUst, MG, MA, Gmat = map(jnp.asarray, (EUst, MG, MA, Gmat))

    C3v = jnp.concatenate([C3[_TGTP, h] for h in range(H)])
    C2v = jnp.concatenate([C2[_SRCP, h] for h in range(H)])
    C4v = jnp.concatenate([C4[:, h] for h in range(H)])
    C1v = jnp.concatenate([jnp.full((NE,), C1[h]) for h in range(H)])
    CE = jnp.stack([C3v, C2v, C4v, C1v], axis=0)

    ccf = (cnode + bo).reshape(NOUT)
    gam = jnp.tile(gamma.astype(f32), NUM_NODES)
    bet = jnp.tile(beta.astype(f32), NUM_NODES)
    V3 = jnp.stack([ccf, gam, bet], axis=0)

    E = np.zeros((NOUT, NUM_NODES), dtype=np.float32)
    for t in range(NUM_NODES):
        E[t * HID:(t + 1) * HID, t] = 1.0
    E = jnp.asarray(E)
    ET = E.T

    bf = jnp.bfloat16
    W1 = jnp.concatenate([WT704, WT704 @ E * (1.0 / HID)], axis=1)
    Sb = jnp.concatenate([S_w, S_w2, UW], axis=0)
    Sbig = jnp.concatenate([Sb, Sb @ E * (1.0 / HID)], axis=1)
    MUC = ((ccf @ E) * (1.0 / HID)).reshape(1, NUM_NODES)

    cvals = jnp.concatenate([C3v, C2v, C4v, C1])
    CER = jnp.broadcast_to(cvals[:, None], (146, 16)).astype(jnp.float32)

    return dict(W1=W1.astype(bf), WGA=WGA.astype(bf), EUst=EUst.astype(bf),
                MG=MG.astype(bf), MA=MA.astype(bf), Gmat=Gmat.astype(bf),
                Sbig=Sbig.astype(bf), CE=CE, MUC=MUC, V3=V3,
                E=E.astype(bf), ET=ET.astype(bf), CER=CER)


def _dot(a, b):
    return jnp.dot(a.astype(jnp.bfloat16), b,
                   preferred_element_type=jnp.float32)


def _dotf(a, b):
    return jnp.dot(a, b, preferred_element_type=jnp.float32)


_P_OHC = np.zeros((NUM_NODES, NUM_CASTES), np.float32)
_P_OHC[np.arange(NUM_NODES), _CASTES] = 1.0
_P_OHT = np.zeros((NE, NUM_NODES), np.float32)
_P_OHT[np.arange(NE), _TGTP] = 1.0
_P_OHS = np.zeros((NE, NUM_NODES), np.float32)
_P_OHS[np.arange(NE), _SRCP] = 1.0
_P_OHE = np.zeros((NE, NUM_CASTES * NUM_CASTES), np.float32)
_P_OHE[np.arange(NE), _ETP] = 1.0
_P_DH = np.zeros((HID, H), np.float32)
for _h in range(H):
    _P_DH[_h * D:(_h + 1) * D, _h] = 1.0
_P_M0 = np.zeros((1, HID), np.float32)
_P_M0[0, :D] = 1.0
_P_M1 = np.zeros((1, HID), np.float32)
_P_M1[0, D:] = 1.0
_P_E = np.zeros((NOUT, NUM_NODES), np.float32)
for _t in range(NUM_NODES):
    _P_E[_t * HID:(_t + 1) * HID, _t] = 1.0
_P_EYE11 = np.eye(NUM_NODES, dtype=np.float32)
_P_ONES64 = np.ones((HID, 1), np.float32)


def _prep_body(wn_ref, ct_ref, wq_ref, wk_ref, wv_ref, wo_ref, ebt_ref,
               bn_ref, bq_ref, bk_ref, bv_ref, bo_ref, gam_ref, bet_ref,
               ohc_ref, oht_ref, ohs_ref, ohe_ref, dh_ref, m0_ref, m1_ref,
               e_ref, eye_ref, ones_ref,
               w1_ref, wga_ref, sbig_ref, cer_ref, muc_ref, v3_ref):
    inv = 1.0 / math.sqrt(D)
    Wn = wn_ref[...]
    W_state = Wn[0:EMB]
    wu = Wn[EMB:EMB + 1]
    W_caste = Wn[EMB + 1:]
    Wq, Wk, Wv, Wo = wq_ref[...], wk_ref[...], wv_ref[...], wo_ref[...]
    Dh = dh_ref[...]
    m0, m1 = m0_ref[...], m1_ref[...]
    oht, ohs = oht_ref[...], ohs_ref[...]
    Ec = e_ref[...]
    cn = _dotf(_dotf(ohc_ref[...], ct_ref[...]), W_caste) + bn_ref[...]
    Qc = _dotf(cn, Wq) + bq_ref[...]
    Kc = _dotf(cn, Wk) + bk_ref[...]
    Vc = _dotf(cn, Wv) + bv_ref[...]
    qu = _dotf(wu, Wq)
    ku = _dotf(wu, Wk)
    vu = _dotf(wu, Wv)
    WQ = _dotf(W_state, Wq)
    aW = _dotf(WQ * ku, Dh) * inv
    gW = jnp.concatenate(
        [_dotf(WQ, (Kc * m0).T) * inv, _dotf(WQ, (Kc * m1).T) * inv],
        axis=1)
    wga_ref[...] = jnp.concatenate([gW, aW], axis=1).astype(jnp.bfloat16)

    C1 = _dotf(qu * ku, Dh) * inv
    C2 = _dotf(Kc * qu, Dh) * inv
    C3 = _dotf(Qc * ku, Dh) * inv
    QcT = _dotf(oht, Qc)
    KcS = _dotf(ohs, Kc)
    C4 = _dotf(QcT * KcS, Dh) * inv + _dotf(ohe_ref[...], ebt_ref[...])
    C3c = _dotf(oht, C3)
    C2c = _dotf(ohs, C2)
    cvals = jnp.concatenate(
        [C3c[:, 0:1], C3c[:, 1:2], C2c[:, 0:1], C2c[:, 1:2],
         C4[:, 0:1], C4[:, 1:2], C1.T], axis=0)
    cer_ref[...] = jnp.broadcast_to(cvals, (146, _SC_LANES))

    P0 = _dotf(vu * m0, Wo)
    P1 = _dotf(vu * m1, Wo)
    Rm0 = _dotf(Vc * m0, Wo)
    Rm1 = _dotf(Vc * m1, Wo)
    VbWo = _dotf(W_state, _dotf(Wv, Wo))
    Wa = W_state + VbWo
    WT704 = jnp.concatenate(
        [Wa if _HAS_IN[t] else W_state for t in range(NUM_NODES)], axis=1)
    w1_ref[...] = jnp.concatenate(
        [WT704, _dotf(WT704, Ec) * (1.0 / HID)], axis=1).astype(jnp.bfloat16)

    Z0 = _dotf(ohs, Rm0)
    Z1 = _dotf(ohs, Rm1)
    S_w0 = jnp.concatenate([Z0 * oht[:, t:t + 1] for t in range(NUM_NODES)], axis=1)
    S_w1 = jnp.concatenate([Z1 * oht[:, t:t + 1] for t in range(NUM_NODES)], axis=1)
    S20 = jnp.concatenate([oht[:, t:t + 1] * P0 for t in range(NUM_NODES)], axis=1)
    S21 = jnp.concatenate([oht[:, t:t + 1] * P1 for t in range(NUM_NODES)], axis=1)
    eye = eye_ref[...]
    UW = jnp.concatenate([eye[:, t:t + 1] * wu for t in range(NUM_NODES)], axis=1)
    Sb = jnp.concatenate([S_w0, S_w1, S20, S21, UW], axis=0)
    sbig_ref[...] = jnp.concatenate(
        [Sb, _dotf(Sb, Ec) * (1.0 / HID)], axis=1).astype(jnp.bfloat16)

    cnbo = cn + bo_ref[...]
    cc_row = jnp.concatenate([cnbo[t:t + 1] for t in range(NUM_NODES)], axis=1)
    gam_row = jnp.concatenate([gam_ref[...]] * NUM_NODES, axis=1)
    bet_row = jnp.concatenate([bet_ref[...]] * NUM_NODES, axis=1)
    v3_ref[...] = jnp.concatenate([cc_row, gam_row, bet_row], axis=0)
    muc_ref[...] = (_dotf(cnbo, ones_ref[...]) * (1.0 / HID)).T


def _prep_pallas(W_node, b_node, caste_table, Wq, bq, Wk, bk, Wv, bv,
                 edge_bias_table, Wo, bo, gamma, beta):
    f32 = jnp.float32
    row = lambda v: v.astype(f32).reshape(1, -1)
    consts = [_P_OHC, _P_OHT, _P_OHS, _P_OHE, _P_DH, _P_M0, _P_M1, _P_E,
              _P_EYE11, _P_ONES64]
    args = ([W_node.astype(f32), caste_table.astype(f32), Wq.astype(f32),
             Wk.astype(f32), Wv.astype(f32), Wo.astype(f32),
             edge_bias_table.astype(f32),
             row(b_node), row(bq), row(bk), row(bv), row(bo),
             row(gamma), row(beta)]
            + [jnp.asarray(c) for c in consts])
    bf = jnp.bfloat16
    out_shapes = (
        jax.ShapeDtypeStruct((EMB, NOUT + NUM_NODES), bf),
        jax.ShapeDtypeStruct((EMB, 2 * NUM_NODES + H), bf),
        jax.ShapeDtypeStruct((107, NOUT + NUM_NODES), bf),
        jax.ShapeDtypeStruct((146, _SC_LANES), f32),
        jax.ShapeDtypeStruct((1, NUM_NODES), f32),
        jax.ShapeDtypeStruct((3, NOUT), f32),
    )
    outs = pl.pallas_call(_prep_body, out_shape=out_shapes)(*args)
    return dict(W1=outs[0], WGA=outs[1], Sbig=outs[2], CER=outs[3],
                MUC=outs[4], V3=outs[5],
                E=jnp.asarray(_P_E).astype(bf),
                ET=jnp.asarray(_P_E.T).astype(bf))


def _tc_body(se_ref, u_ref, W1_ref, WGA_ref, EUst_ref, MG_ref, MA_ref,
             Gmat_ref, Sbig_ref, CE_ref, MUC_ref, V3_ref, E_ref,
             ET_ref, out_ref):
    se = se_ref[...]
    u = u_ref[...]
    CE = CE_ref[...]
    V3 = V3_ref[...]
    X = _dot(se, W1_ref[...])
    GA = _dot(se, WGA_ref[...])
    UU = _dot(u, EUst_ref[...])
    u_src = UU[:, :NSC]
    u_tgt = UU[:, NSC:]
    G48 = _dot(GA, MG_ref[...])
    A48 = _dot(GA, MA_ref[...])
    s = u_src * (A48 + CE[3] * u_tgt + CE[0]) + G48 + CE[1] * u_tgt + CE[2]
    m = jnp.max(s, axis=1, keepdims=True)
    ez = jnp.exp(s - m)
    denom = _dot(ez, Gmat_ref[...])
    w = ez / denom
    cat = jnp.concatenate([w, w * u_src, u], axis=1)
    Y = _dot(cat, Sbig_ref[...])
    y = X[:, :NOUT] + Y[:, :NOUT] + V3[0]
    mu = X[:, NOUT:] + Y[:, NOUT:] + MUC_ref[...][0]
    var = _dot(y * y, E_ref[...]) * (1.0 / HID) - mu * mu
    q = jax.lax.rsqrt(var + 1e-5)
    qf = _dot(q, ET_ref[...])
    muf = _dot(mu * q, ET_ref[...])
    res = (y * qf - muf) * V3[1] + V3[2]
    out_ref[...] = res.reshape(res.shape[0], NUM_NODES, HID)


def _pick_block(Bsz):
    for bb in (2048, 1024, 512, 256, 128, 64, 32, 16, 8):
        if Bsz % bb == 0:
            return bb
    return Bsz


_SC_WORKERS = 32
_SC_LANES = 16


def _sc_weights(u, GA, CER):
    Bsz = u.shape[0]
    per_w = Bsz // _SC_WORKERS
    nstrips = per_w // _SC_LANES
    i32 = jnp.int32
    mesh = plsc.VectorSubcoreMesh(core_axis_name="c", subcore_axis_name="s")

    NGA = 2 * NUM_NODES + H

    @functools.partial(
        pl.kernel, mesh=mesh,
        compiler_params=pltpu.CompilerParams(needs_layout_passes=False),
        out_type=jax.ShapeDtypeStruct((Bsz * NW,), jnp.float32),
        scratch_types=[
            pltpu.VMEM((per_w * NUM_NODES,), jnp.float32),
            pltpu.VMEM((per_w * NGA,), jnp.float32),
            pltpu.VMEM((146 * _SC_LANES,), jnp.float32),
            pltpu.VMEM((per_w * NW,), jnp.float32),
        ])
    def k(u_hbm, ga_hbm, cer_hbm, out_hbm, u_v, ga_v, cer_v, w_v):
        wid = lax.axis_index("s") * 2 + lax.axis_index("c")
        base = wid * per_w
        pltpu.sync_copy(u_hbm.at[pl.ds(base * NUM_NODES, per_w * NUM_NODES)], u_v)
        pltpu.sync_copy(ga_hbm.at[pl.ds(base * NGA, per_w * NGA)], ga_v)
        pltpu.sync_copy(cer_hbm, cer_v)

        def strip(si, carry):
            rows = si * _SC_LANES + lax.iota(i32, _SC_LANES)
            rows_u = rows * NUM_NODES
            rows_g = rows * NGA
            rows_w = rows * NW
            uu = [plsc.load_gather(u_v, [rows_u + n])
                  for n in range(NUM_NODES)]
            gg = [plsc.load_gather(ga_v, [rows_g + n])
                  for n in range(NGA)]
            for h in range(H):
                A_h = gg[2 * NUM_NODES + h]
                c1 = cer_v[pl.ds((144 + h) * _SC_LANES, _SC_LANES)]
                zt = {}
                for t in set(int(x) for x in _TGTP):
                    c3 = cer_v[pl.ds((h * NE + _T2SEG[t]) * _SC_LANES,
                                     _SC_LANES)]
                    zt[t] = A_h + c1 * uu[t] + c3
                sc = []
                for e in range(NE):
                    s_n, t_n = int(_SRCP[e]), int(_TGTP[e])
                    c2 = cer_v[pl.ds((48 + h * NE + e) * _SC_LANES,
                                     _SC_LANES)]
                    c4 = cer_v[pl.ds((96 + h * NE + e) * _SC_LANES,
                                     _SC_LANES)]
                    val = (uu[s_n] * zt[t_n] + gg[h * NUM_NODES + s_n]
                           + c2 * uu[t_n] + c4)
                    sc.append(val)
                for (a, b) in _SEGS:
                    if b - a == 1:
                        w_list = [jnp.full((_SC_LANES,), 1.0, jnp.float32)]
                    else:
                        m = sc[a]
                        for e in range(a + 1, b):
                            m = jnp.maximum(m, sc[e])
                        ex = [jnp.exp(sc[e] - m) for e in range(a, b)]
                        den = ex[0]
                        for x in ex[1:]:
                            den = den + x
                        r = 1.0 / den
                        w_list = [x * r for x in ex]
                    for j, wv in enumerate(w_list):
                        e = a + j
                        plsc.store_scatter(w_v, [rows_w + (h * NE + e)], wv)
                        plsc.store_scatter(
                            w_v, [rows_w + (NSC + h * NE + e)],
                            wv * uu[int(_SRCP[e])])
            return carry

        lax.fori_loop(0, nstrips, strip, 0)
        pltpu.sync_copy(w_v, out_hbm.at[pl.ds(base * NW, per_w * NW)])

    return k(u.reshape(-1), GA.reshape(-1), CER.reshape(-1)).reshape(Bsz, NW)


def _tc1_body(se_ref, WGA_ref, ga_ref):
    ga_ref[...] = _dot(se_ref[...], WGA_ref[...])


def _tc2_body(se_ref, u_ref, w_ref, W1_ref, Sbig_ref, MUC_ref, V3_ref,
              E_ref, ET_ref, out_ref):
    se = se_ref[...]
    u = u_ref[...]
    V3 = V3_ref[...]
    X = _dot(se, W1_ref[...])
    cat = jnp.concatenate([w_ref[...], u], axis=1)
    Y = _dot(cat, Sbig_ref[...])
    y = X[:, :NOUT] + Y[:, :NOUT] + V3[0]
    mu = X[:, NOUT:] + Y[:, NOUT:] + MUC_ref[...][0]
    var = _dot(y * y, E_ref[...]) * (1.0 / HID) - mu * mu
    q = jax.lax.rsqrt(var + 1e-5)
    qf = _dot(q, ET_ref[...])
    muf = _dot(mu * q, ET_ref[...])
    res = (y * qf - muf) * V3[1] + V3[2]
    out_ref[...] = res.reshape(res.shape[0], NUM_NODES, HID)


def _full(shape):
    return pl.BlockSpec(shape, lambda i: tuple(0 for _ in shape))


def _tc_only(se, u, ops, Bsz):
    Bb = _pick_block(Bsz)
    grid = Bsz // Bb
    names = ('W1', 'WGA', 'EUst', 'MG', 'MA', 'Gmat', 'Sbig',
             'CE', 'MUC', 'V3', 'E', 'ET')
    out = pl.pallas_call(
        _tc_body,
        grid=(grid,),
        in_specs=[
            pl.BlockSpec((Bb, EMB), lambda i: (i, 0)),
            pl.BlockSpec((Bb, NUM_NODES), lambda i: (i, 0)),
        ] + [_full(ops[n].shape) for n in names],
        out_specs=pl.BlockSpec((Bb, NUM_NODES, HID), lambda i: (i, 0, 0)),
        out_shape=jax.ShapeDtypeStruct((Bsz, NUM_NODES, HID), jnp.float32),
    )(se, u, *[ops[n] for n in names])
    return out


def _hybrid(se, u, ops, Bsz):
    Bb = _pick_block(Bsz)
    grid = Bsz // Bb
    GA = pl.pallas_call(
        _tc1_body,
        grid=(grid,),
        in_specs=[pl.BlockSpec((Bb, EMB), lambda i: (i, 0)),
                  _full(ops['WGA'].shape)],
        out_specs=pl.BlockSpec((Bb, 2 * NUM_NODES + H), lambda i: (i, 0)),
        out_shape=jax.ShapeDtypeStruct((Bsz, 2 * NUM_NODES + H), jnp.float32),
    )(se, ops['WGA'])
    w96 = _sc_weights(u, GA, ops['CER'])
    names = ('W1', 'Sbig', 'MUC', 'V3', 'E', 'ET')
    out = pl.pallas_call(
        _tc2_body,
        grid=(grid,),
        in_specs=[
            pl.BlockSpec((Bb, EMB), lambda i: (i, 0)),
            pl.BlockSpec((Bb, NUM_NODES), lambda i: (i, 0)),
            pl.BlockSpec((Bb, NW), lambda i: (i, 0)),
        ] + [_full(ops[n].shape) for n in names],
        out_specs=pl.BlockSpec((Bb, NUM_NODES, HID), lambda i: (i, 0, 0)),
        out_shape=jax.ShapeDtypeStruct((Bsz, NUM_NODES, HID), jnp.float32),
    )(se, u, w96, *[ops[n] for n in names])
    return out


def kernel(state_embedding, urgency_vector, signal_summary, W_node, b_node,
           caste_table, Wq, bq, Wk, bk, Wv, bv, edge_bias_table, Wo, bo,
           gamma, beta):
    del signal_summary
    Bsz = state_embedding.shape[0]
    se = state_embedding.astype(jnp.float32)
    u = urgency_vector.astype(jnp.float32)
    if Bsz % (_SC_WORKERS * _SC_LANES) == 0:
        ops = _prep_pallas(W_node, b_node, caste_table, Wq, bq, Wk, bk,
                           Wv, bv, edge_bias_table, Wo, bo, gamma, beta)
        out = _hybrid(se, u, ops, Bsz)
    else:
        ops = _prep(W_node, b_node, caste_table, Wq, bq, Wk, bk, Wv, bv,
                    edge_bias_table, Wo, bo, gamma, beta)
        out = _tc_only(se, u, ops, Bsz)
    return out


# speedup vs baseline: 1.5372x; 1.5372x over previous
import functools
import math

import numpy as np
import jax
import jax.numpy as jnp
from jax import lax
from jax.experimental import pallas as pl
from jax.experimental.pallas import tpu as pltpu
from jax.experimental.pallas import tpu_sc as plsc

NUM_CASTES = 5
NUM_NODES = 11
EMB = 128
HID = 64
H = 2
D = HID // H

_NODE_INDEX = {'atp_executor': 0, 'order_tracking': 1, 'po_creation': 2, 'rebalancing': 3, 'subcontracting': 4, 'safety_stock': 5, 'forecast_adj': 6, 'quality': 7, 'maintenance': 8, 'mo_execution': 9, 'to_execution': 10}
_TO_CASTE = {'atp_executor': 0, 'order_tracking': 0, 'po_creation': 1, 'rebalancing': 1, 'subcontracting': 1, 'safety_stock': 2, 'forecast_adj': 2, 'quality': 3, 'maintenance': 3, 'mo_execution': 4, 'to_execution': 4}
_EDGES = [('atp_executor','po_creation'),('atp_executor','rebalancing'),('atp_executor','subcontracting'),('atp_executor','safety_stock'),('order_tracking','po_creation'),('po_creation','atp_executor'),('po_creation','order_tracking'),('rebalancing','atp_executor'),('rebalancing','to_execution'),('subcontracting','mo_execution'),('safety_stock','po_creation'),('safety_stock','atp_executor'),('forecast_adj','po_creation'),('forecast_adj','safety_stock'),('forecast_adj','atp_executor'),('quality','atp_executor'),('quality','mo_execution'),('maintenance','mo_execution'),('maintenance','subcontracting'),('mo_execution','atp_executor'),('mo_execution','po_creation'),('mo_execution','subcontracting'),('to_execution','order_tracking'),('to_execution','rebalancing')]

_SRC = np.array([_NODE_INDEX[s] for s, t in _EDGES], dtype=np.int32)
_TGT = np.array([_NODE_INDEX[t] for s, t in _EDGES], dtype=np.int32)
_ETYPE = np.array([_TO_CASTE[s] * NUM_CASTES + _TO_CASTE[t] for s, t in _EDGES], dtype=np.int32)
_CASTES = np.array([_TO_CASTE[n] for n in sorted(_NODE_INDEX, key=lambda k: _NODE_INDEX[k])], dtype=np.int32)
NE = len(_EDGES)

_PERM = np.argsort(_TGT, kind='stable')
_SRCP = _SRC[_PERM]
_TGTP = _TGT[_PERM]
_ETP = _ETYPE[_PERM]
_SEGS = []
_i = 0
while _i < NE:
    _j = _i
    while _j < NE and _TGTP[_j] == _TGTP[_i]:
        _j += 1
    _SEGS.append((int(_i), int(_j)))
    _i = _j
_HAS_IN = np.zeros(NUM_NODES, dtype=np.float32)
_HAS_IN[np.unique(_TGTP)] = 1.0
_T2SEG = {int(_TGTP[a]): a for (a, b) in reversed(_SEGS)}

NSC = 2 * NE
NW = 2 * NSC
NOUT = NUM_NODES * HID


def _prep(W_node, b_node, caste_table, Wq, bq, Wk, bk, Wv, bv,
          edge_bias_table, Wo, bo, gamma, beta):
    f32 = jnp.float32
    W_state = W_node[:EMB].astype(f32)
    w_u = W_node[EMB].astype(f32)
    W_caste = W_node[EMB + 1:].astype(f32)
    cnode = caste_table[_CASTES] @ W_caste + b_node
    Qc = (cnode @ Wq + bq).reshape(NUM_NODES, H, D)
    Kc = (cnode @ Wk + bk).reshape(NUM_NODES, H, D)
    Vc = (cnode @ Wv + bv).reshape(NUM_NODES, H, D)
    qu = (w_u @ Wq).reshape(H, D)
    ku = (w_u @ Wk).reshape(H, D)
    vu = (w_u @ Wv).reshape(H, D)
    WQ = (W_state @ Wq).reshape(EMB, H, D)
    inv = 1.0 / math.sqrt(D)

    aW = jnp.einsum('khd,hd->kh', WQ, ku) * inv
    gW = (jnp.einsum('khd,shd->khs', WQ, Kc) * inv).reshape(EMB, NUM_NODES * H)
    C1 = jnp.einsum('hd,hd->h', qu, ku) * inv
    C2 = jnp.einsum('hd,shd->sh', qu, Kc) * inv
    C3 = jnp.einsum('thd,hd->th', Qc, ku) * inv
    C5 = jnp.einsum('thd,shd->tsh', Qc, Kc) * inv
    C4 = C5[_TGTP, _SRCP, :] + edge_bias_table[_ETP]

    Woh = Wo.reshape(H, D, HID)
    P = jnp.einsum('hd,hdo->ho', vu, Woh)
    Rm = jnp.einsum('shd,hdo->sho', Vc, Woh)
    VbWo = W_state @ Wv @ Wo

    WT704 = jnp.concatenate(
        [W_state + _HAS_IN[t] * VbWo for t in range(NUM_NODES)], axis=1)
    WGA = jnp.concatenate([gW, aW], axis=1)

    tgt_oh = np.zeros((NE, NUM_NODES), dtype=np.float32)
    tgt_oh[np.arange(NE), _TGTP] = 1.0
    tgt_oh = jnp.asarray(tgt_oh)
    S_w = jnp.concatenate([
        jnp.einsum('et,eo->eto', tgt_oh, Rm[_SRCP, h]).reshape(NE, NOUT)
        for h in range(H)], axis=0)
    S_w2 = jnp.concatenate([
        jnp.einsum('et,o->eto', tgt_oh, P[h]).reshape(NE, NOUT)
        for h in range(H)], axis=0)

    UW = jnp.einsum('tu,o->tuo', jnp.eye(NUM_NODES, dtype=f32), w_u).reshape(NUM_NODES, NOUT)

    EUst = np.zeros((NUM_NODES, 2 * NSC), dtype=np.float32)
    MG = np.zeros((NUM_NODES * H + H, NSC), dtype=np.float32)
    MA = np.zeros((NUM_NODES * H + H, NSC), dtype=np.float32)
    Gmat = np.zeros((NSC, NSC), dtype=np.float32)
    for h in range(H):
        for e in range(NE):
            c = h * NE + e
            EUst[_SRCP[e], c] = 1.0
            EUst[_TGTP[e], NSC + c] = 1.0
            MG[h * NUM_NODES + _SRCP[e], c] = 1.0
            MA[NUM_NODES * H + h, c] = 1.0
        for (a, b) in _SEGS:
            for e in range(a, b):
                for e2 in range(a, b):
                    Gmat[h * NE + e, h * NE + e2] = 1.0
    EUst, MG, MA, Gmat = map(jnp.asarray, (EUst, MG, MA, Gmat))

    C3v = jnp.concatenate([C3[_TGTP, h] for h in range(H)])
    C2v = jnp.concatenate([C2[_SRCP, h] for h in range(H)])
    C4v = jnp.concatenate([C4[:, h] for h in range(H)])
    C1v = jnp.concatenate([jnp.full((NE,), C1[h]) for h in range(H)])
    CE = jnp.stack([C3v, C2v, C4v, C1v], axis=0)

    ccf = (cnode + bo).reshape(NOUT)
    gam = jnp.tile(gamma.astype(f32), NUM_NODES)
    bet = jnp.tile(beta.astype(f32), NUM_NODES)
    V3 = jnp.stack([ccf, gam, bet], axis=0)

    E = np.zeros((NOUT, NUM_NODES), dtype=np.float32)
    for t in range(NUM_NODES):
        E[t * HID:(t + 1) * HID, t] = 1.0
    E = jnp.asarray(E)
    ET = E.T

    bf = jnp.bfloat16
    W1 = jnp.concatenate([WT704, WT704 @ E * (1.0 / HID)], axis=1)
    Sb = jnp.concatenate([S_w, S_w2, UW], axis=0)
    Sbig = jnp.concatenate([Sb, Sb @ E * (1.0 / HID)], axis=1)
    MUC = ((ccf @ E) * (1.0 / HID)).reshape(1, NUM_NODES)

    cvals = jnp.concatenate([C3v, C2v, C4v, C1])
    CER = jnp.broadcast_to(cvals[:, None], (146, 16)).astype(jnp.float32)

    return dict(W1=W1.astype(bf), WGA=WGA.astype(bf), EUst=EUst.astype(bf),
                MG=MG.astype(bf), MA=MA.astype(bf), Gmat=Gmat.astype(bf),
                Sbig=Sbig.astype(bf), CE=CE, MUC=MUC, V3=V3,
                E=E.astype(bf), ET=ET.astype(bf), CER=CER)


def _dot(a, b):
    return jnp.dot(a.astype(jnp.bfloat16), b,
                   preferred_element_type=jnp.float32)


def _dotf(a, b):
    return jnp.dot(a, b, preferred_element_type=jnp.float32)


_P_OHC = np.zeros((NUM_NODES, NUM_CASTES), np.float32)
_P_OHC[np.arange(NUM_NODES), _CASTES] = 1.0
_P_OHT = np.zeros((NE, NUM_NODES), np.float32)
_P_OHT[np.arange(NE), _TGTP] = 1.0
_P_OHS = np.zeros((NE, NUM_NODES), np.float32)
_P_OHS[np.arange(NE), _SRCP] = 1.0
_P_OHE = np.zeros((NE, NUM_CASTES * NUM_CASTES), np.float32)
_P_OHE[np.arange(NE), _ETP] = 1.0
_P_DH = np.zeros((HID, H), np.float32)
for _h in range(H):
    _P_DH[_h * D:(_h + 1) * D, _h] = 1.0
_P_M0 = np.zeros((1, HID), np.float32)
_P_M0[0, :D] = 1.0
_P_M1 = np.zeros((1, HID), np.float32)
_P_M1[0, D:] = 1.0
_P_E = np.zeros((NOUT, NUM_NODES), np.float32)
for _t in range(NUM_NODES):
    _P_E[_t * HID:(_t + 1) * HID, _t] = 1.0
_P_EYE11 = np.eye(NUM_NODES, dtype=np.float32)
_P_EUS = np.zeros((NUM_NODES, NSC), np.float32)
for _h in range(H):
    for _e in range(NE):
        _P_EUS[_SRCP[_e], _h * NE + _e] = 1.0
_P_ONES64 = np.ones((HID, 1), np.float32)


def _prep_body(wn_ref, ct_ref, wq_ref, wk_ref, wv_ref, wo_ref, ebt_ref,
               bn_ref, bq_ref, bk_ref, bv_ref, bo_ref, gam_ref, bet_ref,
               ohc_ref, oht_ref, ohs_ref, ohe_ref, dh_ref, m0_ref, m1_ref,
               e_ref, eye_ref, ones_ref,
               w1_ref, wga_ref, sbig_ref, cer_ref, muc_ref, v3_ref):
    inv = 1.0 / math.sqrt(D)
    Wn = wn_ref[...]
    W_state = Wn[0:EMB]
    wu = Wn[EMB:EMB + 1]
    W_caste = Wn[EMB + 1:]
    Wq, Wk, Wv, Wo = wq_ref[...], wk_ref[...], wv_ref[...], wo_ref[...]
    Dh = dh_ref[...]
    m0, m1 = m0_ref[...], m1_ref[...]
    oht, ohs = oht_ref[...], ohs_ref[...]
    Ec = e_ref[...]
    cn = _dotf(_dotf(ohc_ref[...], ct_ref[...]), W_caste) + bn_ref[...]
    Qc = _dotf(cn, Wq) + bq_ref[...]
    Kc = _dotf(cn, Wk) + bk_ref[...]
    Vc = _dotf(cn, Wv) + bv_ref[...]
    qu = _dotf(wu, Wq)
    ku = _dotf(wu, Wk)
    vu = _dotf(wu, Wv)
    WQ = _dotf(W_state, Wq)
    aW = _dotf(WQ * ku, Dh) * inv
    gW = jnp.concatenate(
        [_dotf(WQ, (Kc * m0).T) * inv, _dotf(WQ, (Kc * m1).T) * inv],
        axis=1)
    wga_ref[...] = jnp.concatenate([gW, aW], axis=1).astype(jnp.bfloat16)

    C1 = _dotf(qu * ku, Dh) * inv
    C2 = _dotf(Kc * qu, Dh) * inv
    C3 = _dotf(Qc * ku, Dh) * inv
    QcT = _dotf(oht, Qc)
    KcS = _dotf(ohs, Kc)
    C4 = _dotf(QcT * KcS, Dh) * inv + _dotf(ohe_ref[...], ebt_ref[...])
    C3c = _dotf(oht, C3)
    C2c = _dotf(ohs, C2)
    cvals = jnp.concatenate(
        [C3c[:, 0:1], C3c[:, 1:2], C2c[:, 0:1], C2c[:, 1:2],
         C4[:, 0:1], C4[:, 1:2], C1.T], axis=0)
    cer_ref[...] = jnp.broadcast_to(cvals, (146, _SC_LANES))

    P0 = _dotf(vu * m0, Wo)
    P1 = _dotf(vu * m1, Wo)
    Rm0 = _dotf(Vc * m0, Wo)
    Rm1 = _dotf(Vc * m1, Wo)
    VbWo = _dotf(W_state, _dotf(Wv, Wo))
    Wa = W_state + VbWo
    WT704 = jnp.concatenate(
        [Wa if _HAS_IN[t] else W_state for t in range(NUM_NODES)], axis=1)
    w1_ref[...] = jnp.concatenate(
        [WT704, _dotf(WT704, Ec) * (1.0 / HID)], axis=1).astype(jnp.bfloat16)

    Z0 = _dotf(ohs, Rm0)
    Z1 = _dotf(ohs, Rm1)
    S_w0 = jnp.concatenate([Z0 * oht[:, t:t + 1] for t in range(NUM_NODES)], axis=1)
    S_w1 = jnp.concatenate([Z1 * oht[:, t:t + 1] for t in range(NUM_NODES)], axis=1)
    S20 = jnp.concatenate([oht[:, t:t + 1] * P0 for t in range(NUM_NODES)], axis=1)
    S21 = jnp.concatenate([oht[:, t:t + 1] * P1 for t in range(NUM_NODES)], axis=1)
    eye = eye_ref[...]
    UW = jnp.concatenate([eye[:, t:t + 1] * wu for t in range(NUM_NODES)], axis=1)
    Sb = jnp.concatenate([S_w0, S_w1, S20, S21, UW], axis=0)
    sbig_ref[...] = jnp.concatenate(
        [Sb, _dotf(Sb, Ec) * (1.0 / HID)], axis=1).astype(jnp.bfloat16)

    cnbo = cn + bo_ref[...]
    cc_row = jnp.concatenate([cnbo[t:t + 1] for t in range(NUM_NODES)], axis=1)
    gam_row = jnp.concatenate([gam_ref[...]] * NUM_NODES, axis=1)
    bet_row = jnp.concatenate([bet_ref[...]] * NUM_NODES, axis=1)
    v3_ref[...] = jnp.concatenate([cc_row, gam_row, bet_row], axis=0)
    muc_ref[...] = (_dotf(cnbo, ones_ref[...]) * (1.0 / HID)).T


def _prep_pallas(W_node, b_node, caste_table, Wq, bq, Wk, bk, Wv, bv,
                 edge_bias_table, Wo, bo, gamma, beta):
    f32 = jnp.float32
    row = lambda v: v.astype(f32).reshape(1, -1)
    consts = [_P_OHC, _P_OHT, _P_OHS, _P_OHE, _P_DH, _P_M0, _P_M1, _P_E,
              _P_EYE11, _P_ONES64]
    args = ([W_node.astype(f32), caste_table.astype(f32), Wq.astype(f32),
             Wk.astype(f32), Wv.astype(f32), Wo.astype(f32),
             edge_bias_table.astype(f32),
             row(b_node), row(bq), row(bk), row(bv), row(bo),
             row(gamma), row(beta)]
            + [jnp.asarray(c) for c in consts])
    bf = jnp.bfloat16
    out_shapes = (
        jax.ShapeDtypeStruct((EMB, NOUT + NUM_NODES), bf),
        jax.ShapeDtypeStruct((EMB, 2 * NUM_NODES + H), bf),
        jax.ShapeDtypeStruct((107, NOUT + NUM_NODES), bf),
        jax.ShapeDtypeStruct((146, _SC_LANES), f32),
        jax.ShapeDtypeStruct((1, NUM_NODES), f32),
        jax.ShapeDtypeStruct((3, NOUT), f32),
    )
    outs = pl.pallas_call(_prep_body, out_shape=out_shapes)(*args)
    return dict(W1=outs[0], WGA=outs[1], Sbig=outs[2], CER=outs[3],
                MUC=outs[4], V3=outs[5],
                E=jnp.asarray(_P_E).astype(bf),
                ET=jnp.asarray(_P_E.T).astype(bf),
                EUS=jnp.asarray(_P_EUS).astype(bf))


def _tc_body(se_ref, u_ref, W1_ref, WGA_ref, EUst_ref, MG_ref, MA_ref,
             Gmat_ref, Sbig_ref, CE_ref, MUC_ref, V3_ref, E_ref,
             ET_ref, out_ref):
    se = se_ref[...]
    u = u_ref[...]
    CE = CE_ref[...]
    V3 = V3_ref[...]
    X = _dot(se, W1_ref[...])
    GA = _dot(se, WGA_ref[...])
    UU = _dot(u, EUst_ref[...])
    u_src = UU[:, :NSC]
    u_tgt = UU[:, NSC:]
    G48 = _dot(GA, MG_ref[...])
    A48 = _dot(GA, MA_ref[...])
    s = u_src * (A48 + CE[3] * u_tgt + CE[0]) + G48 + CE[1] * u_tgt + CE[2]
    m = jnp.max(s, axis=1, keepdims=True)
    ez = jnp.exp(s - m)
    denom = _dot(ez, Gmat_ref[...])
    w = ez / denom
    cat = jnp.concatenate([w, w * u_src, u], axis=1)
    Y = _dot(cat, Sbig_ref[...])
    y = X[:, :NOUT] + Y[:, :NOUT] + V3[0]
    mu = X[:, NOUT:] + Y[:, NOUT:] + MUC_ref[...][0]
    var = _dot(y * y, E_ref[...]) * (1.0 / HID) - mu * mu
    q = jax.lax.rsqrt(var + 1e-5)
    qf = _dot(q, ET_ref[...])
    muf = _dot(mu * q, ET_ref[...])
    out_ref[...] = (y * qf - muf) * V3[1] + V3[2]


def _pick_block(Bsz):
    for bb in (2048, 1024, 512, 256, 128, 64, 32, 16, 8):
        if Bsz % bb == 0:
            return bb
    return Bsz


_SC_WORKERS = 32
_SC_LANES = 16


def _sc_weights(u, GA, CER):
    Bsz = u.shape[0]
    per_w = Bsz // _SC_WORKERS
    nstrips = per_w // _SC_LANES
    i32 = jnp.int32
    mesh = plsc.VectorSubcoreMesh(core_axis_name="c", subcore_axis_name="s")

    NGA = 2 * NUM_NODES + H

    @functools.partial(
        pl.kernel, mesh=mesh,
        compiler_params=pltpu.CompilerParams(needs_layout_passes=False),
        out_type=jax.ShapeDtypeStruct((Bsz * NSC,), jnp.float32),
        scratch_types=[
            pltpu.VMEM((per_w * NUM_NODES,), jnp.float32),
            pltpu.VMEM((per_w * NGA,), jnp.float32),
            pltpu.VMEM((146 * _SC_LANES,), jnp.float32),
            pltpu.VMEM((per_w * NSC,), jnp.float32),
        ])
    def k(u_hbm, ga_hbm, cer_hbm, out_hbm, u_v, ga_v, cer_v, w_v):
        wid = lax.axis_index("s") * 2 + lax.axis_index("c")
        base = wid * per_w
        pltpu.sync_copy(u_hbm.at[pl.ds(base * NUM_NODES, per_w * NUM_NODES)], u_v)
        pltpu.sync_copy(ga_hbm.at[pl.ds(base * NGA, per_w * NGA)], ga_v)
        pltpu.sync_copy(cer_hbm, cer_v)

        def strip(si, carry):
            rows = si * _SC_LANES + lax.iota(i32, _SC_LANES)
            rows_u = rows * NUM_NODES
            rows_g = rows * NGA
            rows_w = rows * NSC
            uu = [plsc.load_gather(u_v, [rows_u + n])
                  for n in range(NUM_NODES)]
            gg = [plsc.load_gather(ga_v, [rows_g + n])
                  for n in range(NGA)]
            for h in range(H):
                A_h = gg[2 * NUM_NODES + h]
                c1 = cer_v[pl.ds((144 + h) * _SC_LANES, _SC_LANES)]
                zt = {}
                for t in set(int(x) for x in _TGTP):
                    c3 = cer_v[pl.ds((h * NE + _T2SEG[t]) * _SC_LANES,
                                     _SC_LANES)]
                    zt[t] = A_h + c1 * uu[t] + c3
                sc = []
                for e in range(NE):
                    s_n, t_n = int(_SRCP[e]), int(_TGTP[e])
                    c2 = cer_v[pl.ds((48 + h * NE + e) * _SC_LANES,
                                     _SC_LANES)]
                    c4 = cer_v[pl.ds((96 + h * NE + e) * _SC_LANES,
                                     _SC_LANES)]
                    val = (uu[s_n] * zt[t_n] + gg[h * NUM_NODES + s_n]
                           + c2 * uu[t_n] + c4)
                    sc.append(val)
                for (a, b) in _SEGS:
                    if b - a == 1:
                        w_list = [jnp.full((_SC_LANES,), 1.0, jnp.float32)]
                    else:
                        m = sc[a]
                        for e in range(a + 1, b):
                            m = jnp.maximum(m, sc[e])
                        ex = [jnp.exp(sc[e] - m) for e in range(a, b)]
                        den = ex[0]
                        for x in ex[1:]:
                            den = den + x
                        r = 1.0 / den
                        w_list = [x * r for x in ex]
                    for j, wv in enumerate(w_list):
                        e = a + j
                        plsc.store_scatter(w_v, [rows_w + (h * NE + e)], wv)
            return carry

        lax.fori_loop(0, nstrips, strip, 0)
        pltpu.sync_copy(w_v, out_hbm.at[pl.ds(base * NSC, per_w * NSC)])

    return k(u.reshape(-1), GA.reshape(-1), CER.reshape(-1)).reshape(Bsz, NSC)


def _tc1_body(se_ref, WGA_ref, ga_ref):
    ga_ref[...] = _dot(se_ref[...], WGA_ref[...])


def _tc2_body(se_ref, u_ref, w_ref, W1_ref, Sbig_ref, EUS_ref, MUC_ref,
              V3_ref, E_ref, ET_ref, out_ref):
    se = se_ref[...]
    u = u_ref[...]
    V3 = V3_ref[...]
    X = _dot(se, W1_ref[...])
    w = w_ref[...]
    w2 = w * _dot(u, EUS_ref[...])
    cat = jnp.concatenate([w, w2, u], axis=1)
    Y = _dot(cat, Sbig_ref[...])
    y = X[:, :NOUT] + Y[:, :NOUT] + V3[0]
    mu = X[:, NOUT:] + Y[:, NOUT:] + MUC_ref[...][0]
    var = _dot(y * y, E_ref[...]) * (1.0 / HID) - mu * mu
    q = jax.lax.rsqrt(var + 1e-5)
    qf = _dot(q, ET_ref[...])
    muf = _dot(mu * q, ET_ref[...])
    out_ref[...] = (y * qf - muf) * V3[1] + V3[2]


def _full(shape):
    return pl.BlockSpec(shape, lambda i: tuple(0 for _ in shape))


def _tc_only(se, u, ops, Bsz):
    Bb = _pick_block(Bsz)
    grid = Bsz // Bb
    names = ('W1', 'WGA', 'EUst', 'MG', 'MA', 'Gmat', 'Sbig',
             'CE', 'MUC', 'V3', 'E', 'ET')
    out = pl.pallas_call(
        _tc_body,
        grid=(grid,),
        in_specs=[
            pl.BlockSpec((Bb, EMB), lambda i: (i, 0)),
            pl.BlockSpec((Bb, NUM_NODES), lambda i: (i, 0)),
        ] + [_full(ops[n].shape) for n in names],
        out_specs=pl.BlockSpec((Bb, NOUT), lambda i: (i, 0)),
        out_shape=jax.ShapeDtypeStruct((Bsz, NOUT), jnp.float32),
    )(se, u, *[ops[n] for n in names])
    return out


def _hybrid(se, u, ops, Bsz):
    Bb = _pick_block(Bsz)
    grid = Bsz // Bb
    GA = pl.pallas_call(
        _tc1_body,
        grid=(grid,),
        in_specs=[pl.BlockSpec((Bb, EMB), lambda i: (i, 0)),
                  _full(ops['WGA'].shape)],
        out_specs=pl.BlockSpec((Bb, 2 * NUM_NODES + H), lambda i: (i, 0)),
        out_shape=jax.ShapeDtypeStruct((Bsz, 2 * NUM_NODES + H), jnp.float32),
    )(se, ops['WGA'])
    w48 = _sc_weights(u, GA, ops['CER'])
    names = ('W1', 'Sbig', 'EUS', 'MUC', 'V3', 'E', 'ET')
    out = pl.pallas_call(
        _tc2_body,
        grid=(grid,),
        in_specs=[
            pl.BlockSpec((Bb, EMB), lambda i: (i, 0)),
            pl.BlockSpec((Bb, NUM_NODES), lambda i: (i, 0)),
            pl.BlockSpec((Bb, NSC), lambda i: (i, 0)),
        ] + [_full(ops[n].shape) for n in names],
        out_specs=pl.BlockSpec((Bb, NOUT), lambda i: (i, 0)),
        out_shape=jax.ShapeDtypeStruct((Bsz, NOUT), jnp.float32),
    )(se, u, w48, *[ops[n] for n in names])
    return out


def kernel(state_embedding, urgency_vector, signal_summary, W_node, b_node,
           caste_table, Wq, bq, Wk, bk, Wv, bv, edge_bias_table, Wo, bo,
           gamma, beta):
    del signal_summary
    Bsz = state_embedding.shape[0]
    se = state_embedding.astype(jnp.float32)
    u = urgency_vector.astype(jnp.float32)
    if Bsz % (_SC_WORKERS * _SC_LANES) == 0:
        ops = _prep_pallas(W_node, b_node, caste_table, Wq, bq, Wk, bk,
                           Wv, bv, edge_bias_table, Wo, bo, gamma, beta)
        out = _hybrid(se, u, ops, Bsz)
    else:
        ops = _prep(W_node, b_node, caste_table, Wq, bq, Wk, bk, Wv, bv,
                    edge_bias_table, Wo, bo, gamma, beta)
        out = _tc_only(se, u, ops, Bsz)
    return out.reshape(Bsz, NUM_NODES, HID)
